# Initial kernel scaffold; baseline (speedup 1.0000x reference)
import jax
import jax.numpy as jnp
from jax.experimental import pallas as pl


def kernel(x, adj, batch_indices, time_indices, indices):
    raise NotImplementedError("write your pallas kernel here")


# speedup vs baseline: 2.7790x; 2.7790x over previous
import functools

import jax
import jax.numpy as jnp
from jax import lax
from jax.experimental import pallas as pl
from jax.experimental.pallas import tpu as pltpu
from jax.experimental.pallas import tpu_sc as plsc

N_NODES = 10000
D = 128
K = 16
T = 2
NTILES = 16
ROWS_PER_TILE = 10000
CHUNK = 128
NCHUNK = 79
PAD_ROWS = NCHUNK * CHUNK
PASS_N = 5120
ACC_ROWS = PASS_N + 128
SLAB = ACC_ROWS // NTILES
CNT_ROWS = 10496
CSLAB = CNT_ROWS // NTILES
DUMP = N_NODES
CACHE_ROWS = T * NTILES * NCHUNK * CHUNK


def _sc_aggregate(x_flat, adj_b, gidx_p, sidx_p):
  mesh = plsc.VectorSubcoreMesh(core_axis_name="c", subcore_axis_name="s")

  @functools.partial(
      pl.kernel,
      mesh=mesh,
      out_type=[
          jax.ShapeDtypeStruct((2 * T * ACC_ROWS, D), jnp.float32),
          jax.ShapeDtypeStruct((T * CNT_ROWS,), jnp.float32),
          jax.ShapeDtypeStruct((CACHE_ROWS, D), jnp.float32),
      ],
      scratch_types=[
          pltpu.VMEM((CHUNK, D), jnp.float32),
          pltpu.VMEM((CHUNK, D), jnp.float32),
          pltpu.VMEM((CHUNK,), jnp.int32),
          pltpu.VMEM((CHUNK,), jnp.int32),
          pltpu.VMEM((CHUNK,), jnp.int32),
          pltpu.VMEM((CHUNK,), jnp.float32),
          pltpu.VMEM((CSLAB,), jnp.float32),
          pltpu.VMEM((K * K * 16,), jnp.float32),
          pltpu.VMEM_SHARED((ACC_ROWS, D), jnp.float32),
          pltpu.VMEM_SHARED((CNT_ROWS,), jnp.float32),
      ],
  )
  def k(x_hbm, adjb_hbm, gidx_hbm, sidx_hbm, out_hbm, cnt_hbm, cache_hbm,
        g_v, nn_v, gi_v, si_v, sr_v, ones_v, cs_v, adj_v, acc_sp, cnt_sp):
    t = lax.axis_index("c")
    w = lax.axis_index("s")

    pltpu.sync_copy(adjb_hbm, adj_v)
    lane = lax.iota(jnp.int32, 16)

    for p in range(2):
      def zrow(r, carry):
        for dv in range(D // 16):
          g_v[r, pl.ds(dv * 16, 16)] = jnp.zeros((16,), jnp.float32)
        return carry
      lax.fori_loop(0, CHUNK, zrow, 0)
      for q, sz in ((0, 128), (1, 128), (2, 72)):
        pltpu.sync_copy(g_v.at[pl.ds(0, sz)],
                        acc_sp.at[pl.ds(w * SLAB + q * CHUNK, sz)])
      if p == 0:
        def zc(r, carry):
          cs_v[pl.ds(r * 16, 16)] = jnp.zeros((16,), jnp.float32)
          return carry
        lax.fori_loop(0, CSLAB // 16, zc, 0)
        pltpu.sync_copy(cs_v, cnt_sp.at[pl.ds(w * CSLAB, CSLAB)])
        for q in range(CHUNK // 16):
          ones_v[pl.ds(q * 16, 16)] = jnp.ones((16,), jnp.float32)
      plsc.subcore_barrier()

      def chunk_body(c, carry):
        base = ((t * NTILES + w) * NCHUNK + c) * CHUNK
        pltpu.sync_copy(sidx_hbm.at[pl.ds(base, CHUNK)], si_v)

        lo = jnp.int32(p * PASS_N)
        for q in range(CHUNK // 16):
          v = si_v[pl.ds(q * 16, 16)]
          inr = jnp.logical_and(v >= lo, v < lo + PASS_N)
          sr_v[pl.ds(q * 16, 16)] = jnp.where(inr, v - lo, PASS_N + lane)

        if p == 0:
          pltpu.sync_copy(gidx_hbm.at[pl.ds(base, CHUNK)], gi_v)
          pltpu.sync_copy(x_hbm.at[gi_v], g_v)

          for ip in range(K // 2):
            a0 = [adj_v[pl.ds(((2 * ip) * K + j) * 16, 16)] for j in range(K)]
            a1 = [adj_v[pl.ds(((2 * ip + 1) * K + j) * 16, 16)]
                  for j in range(K)]

            def mbody(m, carry2):
              def dvbody(dv, carry3):
                s = pl.ds(dv * 16, 16)
                gs = [g_v[m * K + j, s] for j in range(K)]
                acc0 = a0[0] * gs[0]
                acc1 = a1[0] * gs[0]
                for j in range(1, K):
                  acc0 = acc0 + a0[j] * gs[j]
                  acc1 = acc1 + a1[j] * gs[j]
                nn_v[m * K + 2 * ip, s] = acc0
                nn_v[m * K + 2 * ip + 1, s] = acc1
                return carry3
              return lax.fori_loop(0, D // 16, dvbody, carry2)
            lax.fori_loop(0, CHUNK // K, mbody, 0)

          pltpu.sync_copy(nn_v, cache_hbm.at[pl.ds(base, CHUNK)])
        else:
          pltpu.sync_copy(cache_hbm.at[pl.ds(base, CHUNK)], nn_v)

        pltpu.sync_copy(nn_v, acc_sp.at[sr_v], add=True)
        if p == 0:
          pltpu.sync_copy(ones_v, cnt_sp.at[si_v], add=True)
        return carry
      lax.fori_loop(0, NCHUNK, chunk_body, 0)
      plsc.subcore_barrier()

      obase = (p * T + t) * ACC_ROWS + w * SLAB
      for q, sz in ((0, 128), (1, 128), (2, 72)):
        pltpu.sync_copy(acc_sp.at[pl.ds(w * SLAB + q * CHUNK, sz)],
                        g_v.at[pl.ds(0, sz)])
        pltpu.sync_copy(g_v.at[pl.ds(0, sz)],
                        out_hbm.at[pl.ds(obase + q * CHUNK, sz)])
      if p == 0:
        pltpu.sync_copy(cnt_sp.at[pl.ds(w * CSLAB, CSLAB)], cs_v)
        pltpu.sync_copy(cs_v, cnt_hbm.at[pl.ds(t * CNT_ROWS + w * CSLAB,
                                               CSLAB)])
        plsc.subcore_barrier()

  return k(x_flat, adj_b, gidx_p, sidx_p)


def _div_kernel(acc_ref, cnt_ref, o_ref):
  c = cnt_ref[...] + jnp.float32(10.0 ** (-14))
  o_ref[...] = acc_ref[...] / c


def kernel(x, adj, batch_indices, time_indices, indices):
  B, Tt, N, Dd = x.shape
  _, _, M, Kk = indices.shape

  x_flat = x.reshape(B * Tt * N, Dd)
  flat_g = (batch_indices * Tt + time_indices) * N + indices
  gidx = flat_g.reshape(Tt, NTILES, ROWS_PER_TILE)
  sidx = indices.reshape(Tt, NTILES, ROWS_PER_TILE)
  pad = PAD_ROWS - ROWS_PER_TILE
  gidx_p = jnp.pad(gidx, ((0, 0), (0, 0), (0, pad))).reshape(-1)
  sidx_p = jnp.pad(sidx, ((0, 0), (0, 0), (0, pad)),
                   constant_values=DUMP).reshape(-1)
  adj_b = jnp.broadcast_to(adj[:, :, None], (Kk, Kk, 16)).reshape(-1)
  adj_b = adj_b.astype(jnp.float32)

  acc, cnt, _ = _sc_aggregate(x_flat, adj_b, gidx_p, sidx_p)
  acc = acc.reshape(2, Tt, ACC_ROWS, D)
  acc_full = jnp.concatenate([acc[0, :, :PASS_N], acc[1, :, :PASS_N]], axis=1)
  cnt_full = cnt.reshape(Tt, CNT_ROWS)[:, :2 * PASS_N]

  out = pl.pallas_call(
      _div_kernel,
      grid=(Tt,),
      in_specs=[
          pl.BlockSpec((1, 2 * PASS_N, D), lambda i: (i, 0, 0)),
          pl.BlockSpec((1, 2 * PASS_N, 1), lambda i: (i, 0, 0)),
      ],
      out_specs=pl.BlockSpec((1, 2 * PASS_N, D), lambda i: (i, 0, 0)),
      out_shape=jax.ShapeDtypeStruct((Tt, 2 * PASS_N, D), jnp.float32),
  )(acc_full, cnt_full.reshape(Tt, 2 * PASS_N, 1))

  return out[:, :N].reshape(B, Tt, N, Dd)


# speedup vs baseline: 3.0218x; 1.0873x over previous
import functools

import jax
import jax.numpy as jnp
from jax import lax
from jax.experimental import pallas as pl
from jax.experimental.pallas import tpu as pltpu
from jax.experimental.pallas import tpu_sc as plsc

N_NODES = 10000
D = 128
K = 16
T = 2
NTILES = 16
ROWS_PER_TILE = 10000
CHUNK = 128
NCHUNK = 80
PAD_ROWS = NCHUNK * CHUNK
PASS_N = 5120
ACC_ROWS = PASS_N + 128
SLAB = ACC_ROWS // NTILES
CNT_ROWS = 10496
CSLAB = CNT_ROWS // NTILES
DUMP = N_NODES
CACHE_ROWS = T * NTILES * NCHUNK * CHUNK


def _sc_aggregate(x_flat, adj_b, gidx_p, sidx_p):
  mesh = plsc.VectorSubcoreMesh(core_axis_name="c", subcore_axis_name="s")

  @functools.partial(
      pl.kernel,
      mesh=mesh,
      out_type=[
          jax.ShapeDtypeStruct((2 * T * ACC_ROWS, D), jnp.float32),
          jax.ShapeDtypeStruct((T * CNT_ROWS,), jnp.float32),
          jax.ShapeDtypeStruct((CACHE_ROWS, D), jnp.float32),
      ],
      scratch_types=[
          [pltpu.VMEM((CHUNK, D), jnp.float32) for _ in range(2)],
          [pltpu.VMEM((CHUNK, D), jnp.float32) for _ in range(2)],
          [pltpu.VMEM((CHUNK,), jnp.int32) for _ in range(2)],
          [pltpu.VMEM((CHUNK,), jnp.int32) for _ in range(2)],
          [pltpu.VMEM((CHUNK,), jnp.int32) for _ in range(2)],
          pltpu.VMEM((CHUNK,), jnp.float32),
          pltpu.VMEM((CSLAB,), jnp.float32),
          pltpu.VMEM((K * K * 16,), jnp.float32),
          pltpu.VMEM_SHARED((ACC_ROWS, D), jnp.float32),
          pltpu.VMEM_SHARED((CNT_ROWS,), jnp.float32),
          [pltpu.SemaphoreType.DMA for _ in range(2)],
          [pltpu.SemaphoreType.DMA for _ in range(2)],
          [pltpu.SemaphoreType.DMA for _ in range(2)],
          [pltpu.SemaphoreType.DMA for _ in range(2)],
      ],
  )
  def k(x_hbm, adjb_hbm, gidx_hbm, sidx_hbm, out_hbm, cnt_hbm, cache_hbm,
        g_v, nn_v, gi_v, si_v, sr_v, ones_v, cs_v, adj_v, acc_sp, cnt_sp,
        gsem, csem, ssem, cntsem):
    t = lax.axis_index("c")
    w = lax.axis_index("s")

    pltpu.sync_copy(adjb_hbm, adj_v)
    lane = lax.iota(jnp.int32, 16)
    tbase = (t * NTILES + w) * NCHUNK * CHUNK

    def load_idx(c, b):
      base = tbase + c * CHUNK
      pltpu.sync_copy(gidx_hbm.at[pl.ds(base, CHUNK)], gi_v[b])
      pltpu.sync_copy(sidx_hbm.at[pl.ds(base, CHUNK)], si_v[b])

    def remap(p, b):
      lo = jnp.int32(p * PASS_N)
      for q in range(CHUNK // 16):
        v = si_v[b][pl.ds(q * 16, 16)]
        inr = jnp.logical_and(v >= lo, v < lo + PASS_N)
        sr_v[b][pl.ds(q * 16, 16)] = jnp.where(inr, v - lo, PASS_N + lane)

    def compute(b):
      for ip in range(K // 2):
        a0 = [adj_v[pl.ds(((2 * ip) * K + j) * 16, 16)] for j in range(K)]
        a1 = [adj_v[pl.ds(((2 * ip + 1) * K + j) * 16, 16)]
              for j in range(K)]

        def mbody(m, carry2):
          def dvbody(dv, carry3):
            s = pl.ds(dv * 16, 16)
            gs = [g_v[b][m * K + j, s] for j in range(K)]
            acc0 = a0[0] * gs[0]
            acc1 = a1[0] * gs[0]
            for j in range(1, K):
              acc0 = acc0 + a0[j] * gs[j]
              acc1 = acc1 + a1[j] * gs[j]
            nn_v[b][m * K + 2 * ip, s] = acc0
            nn_v[b][m * K + 2 * ip + 1, s] = acc1
            return carry3
          return lax.fori_loop(0, D // 16, dvbody, carry2)
        lax.fori_loop(0, CHUNK // K, mbody, 0)

    for p in range(2):
      def zrow(r, carry):
        for dv in range(D // 16):
          g_v[0][r, pl.ds(dv * 16, 16)] = jnp.zeros((16,), jnp.float32)
        return carry
      lax.fori_loop(0, CHUNK, zrow, 0)
      for q, sz in ((0, 128), (1, 128), (2, 72)):
        pltpu.sync_copy(g_v[0].at[pl.ds(0, sz)],
                        acc_sp.at[pl.ds(w * SLAB + q * CHUNK, sz)])
      if p == 0:
        def zc(r, carry):
          cs_v[pl.ds(r * 16, 16)] = jnp.zeros((16,), jnp.float32)
          return carry
        lax.fori_loop(0, CSLAB // 16, zc, 0)
        pltpu.sync_copy(cs_v, cnt_sp.at[pl.ds(w * CSLAB, CSLAB)])
        for q in range(CHUNK // 16):
          ones_v[pl.ds(q * 16, 16)] = jnp.ones((16,), jnp.float32)
      plsc.subcore_barrier()

      if p == 0:
        load_idx(0, 0)
        pltpu.async_copy(x_hbm.at[gi_v[0]], g_v[0], gsem[0])

        def half(i, c, b):
          b1 = b ^ 1
          cbase = tbase + c * CHUNK

          @pl.when(c >= 2)
          def _():
            pltpu.make_async_copy(
                nn_v[b], cache_hbm.at[pl.ds(cbase - 2 * CHUNK, CHUNK)],
                csem[b]).wait()
            pltpu.make_async_copy(nn_v[b], acc_sp.at[sr_v[b]], ssem[b]).wait()

          @pl.when(c >= 1)
          def _():
            pltpu.make_async_copy(ones_v, cnt_sp.at[si_v[b1]],
                                  cntsem[b1]).wait()

          @pl.when(c + 1 < NCHUNK)
          def _():
            load_idx(c + 1, b1)
            pltpu.async_copy(x_hbm.at[gi_v[b1]], g_v[b1], gsem[b1])

          pltpu.make_async_copy(x_hbm.at[gi_v[b]], g_v[b], gsem[b]).wait()
          remap(0, b)
          compute(b)
          pltpu.async_copy(nn_v[b], cache_hbm.at[pl.ds(cbase, CHUNK)],
                           csem[b])
          pltpu.async_copy(nn_v[b], acc_sp.at[sr_v[b]], ssem[b], add=True)
          pltpu.async_copy(ones_v, cnt_sp.at[si_v[b]], cntsem[b], add=True)

        def pair(i, carry):
          half(i, 2 * i, 0)
          half(i, 2 * i + 1, 1)
          return carry
        lax.fori_loop(0, NCHUNK // 2, pair, 0)

        for b, c in ((0, NCHUNK - 2), (1, NCHUNK - 1)):
          cbase = tbase + c * CHUNK
          pltpu.make_async_copy(nn_v[b], cache_hbm.at[pl.ds(cbase, CHUNK)],
                                csem[b]).wait()
          pltpu.make_async_copy(nn_v[b], acc_sp.at[sr_v[b]], ssem[b]).wait()
        pltpu.make_async_copy(ones_v, cnt_sp.at[si_v[1]], cntsem[1]).wait()
      else:
        pltpu.async_copy(cache_hbm.at[pl.ds(tbase, CHUNK)], nn_v[0], gsem[0])

        def half1(i, c, b):
          b1 = b ^ 1
          cbase = tbase + c * CHUNK

          @pl.when(c >= 1)
          def _():
            pltpu.make_async_copy(nn_v[b1], acc_sp.at[sr_v[b1]],
                                  ssem[b1]).wait()

          @pl.when(c + 1 < NCHUNK)
          def _():
            pltpu.async_copy(cache_hbm.at[pl.ds(cbase + CHUNK, CHUNK)],
                             nn_v[b1], gsem[b1])

          base = tbase + c * CHUNK
          pltpu.sync_copy(sidx_hbm.at[pl.ds(base, CHUNK)], si_v[b])
          remap(1, b)
          pltpu.make_async_copy(cache_hbm.at[pl.ds(cbase, CHUNK)], nn_v[b],
                                gsem[b]).wait()
          pltpu.async_copy(nn_v[b], acc_sp.at[sr_v[b]], ssem[b], add=True)

        def pair1(i, carry):
          half1(i, 2 * i, 0)
          half1(i, 2 * i + 1, 1)
          return carry
        lax.fori_loop(0, NCHUNK // 2, pair1, 0)
        pltpu.make_async_copy(nn_v[1], acc_sp.at[sr_v[1]], ssem[1]).wait()

      plsc.subcore_barrier()

      obase = (p * T + t) * ACC_ROWS + w * SLAB
      for q, sz in ((0, 128), (1, 128), (2, 72)):
        pltpu.sync_copy(acc_sp.at[pl.ds(w * SLAB + q * CHUNK, sz)],
                        g_v[0].at[pl.ds(0, sz)])
        pltpu.sync_copy(g_v[0].at[pl.ds(0, sz)],
                        out_hbm.at[pl.ds(obase + q * CHUNK, sz)])
      if p == 0:
        pltpu.sync_copy(cnt_sp.at[pl.ds(w * CSLAB, CSLAB)], cs_v)
        pltpu.sync_copy(cs_v, cnt_hbm.at[pl.ds(t * CNT_ROWS + w * CSLAB,
                                               CSLAB)])
        plsc.subcore_barrier()

  return k(x_flat, adj_b, gidx_p, sidx_p)


def _div_kernel(acc_ref, cnt_ref, o_ref):
  c = cnt_ref[...] + jnp.float32(10.0 ** (-14))
  o_ref[...] = acc_ref[...] / c


def kernel(x, adj, batch_indices, time_indices, indices):
  B, Tt, N, Dd = x.shape
  _, _, M, Kk = indices.shape

  x_flat = x.reshape(B * Tt * N, Dd)
  flat_g = (batch_indices * Tt + time_indices) * N + indices
  gidx = flat_g.reshape(Tt, NTILES, ROWS_PER_TILE)
  sidx = indices.reshape(Tt, NTILES, ROWS_PER_TILE)
  pad = PAD_ROWS - ROWS_PER_TILE
  gidx_p = jnp.pad(gidx, ((0, 0), (0, 0), (0, pad))).reshape(-1)
  sidx_p = jnp.pad(sidx, ((0, 0), (0, 0), (0, pad)),
                   constant_values=DUMP).reshape(-1)
  adj_b = jnp.broadcast_to(adj[:, :, None], (Kk, Kk, 16)).reshape(-1)
  adj_b = adj_b.astype(jnp.float32)

  acc, cnt, _ = _sc_aggregate(x_flat, adj_b, gidx_p, sidx_p)
  acc = acc.reshape(2, Tt, ACC_ROWS, D)
  acc_full = jnp.concatenate([acc[0, :, :PASS_N], acc[1, :, :PASS_N]], axis=1)
  cnt_full = cnt.reshape(Tt, CNT_ROWS)[:, :2 * PASS_N]

  out = pl.pallas_call(
      _div_kernel,
      grid=(Tt,),
      in_specs=[
          pl.BlockSpec((1, 2 * PASS_N, D), lambda i: (i, 0, 0)),
          pl.BlockSpec((1, 2 * PASS_N, 1), lambda i: (i, 0, 0)),
      ],
      out_specs=pl.BlockSpec((1, 2 * PASS_N, D), lambda i: (i, 0, 0)),
      out_shape=jax.ShapeDtypeStruct((Tt, 2 * PASS_N, D), jnp.float32),
  )(acc_full, cnt_full.reshape(Tt, 2 * PASS_N, 1))

  return out[:, :N].reshape(B, Tt, N, Dd)
